# baseline (speedup 1.0000x reference)
import functools

import jax
import jax.numpy as jnp
from jax import lax
from jax.experimental import pallas as pl
from jax.experimental.pallas import tpu as pltpu
from jax.experimental.pallas import tpu_sc as plsc

N = 10000
NPAD = 10240
E = 320000
D = 128
NC = 2
NS = 16
NW = NC * NS
EK = 80
ROWS_W = E // (NW * EK)
EPT = ROWS_W * EK
RPT = NPAD // NS
BN = 400

_sc_mesh = plsc.VectorSubcoreMesh(core_axis_name="c", subcore_axis_name="s")


DCH = E // 128
DCB = DCH // NW


@functools.partial(
    pl.kernel,
    mesh=_sc_mesh,
    out_type=jax.ShapeDtypeStruct((NC, NPAD), jnp.float32),
    scratch_types=[
        pltpu.VMEM((128,), jnp.int32),
        pltpu.VMEM((128,), jnp.int32),
        pltpu.VMEM((128,), jnp.float32),
        pltpu.VMEM((RPT,), jnp.float32),
        pltpu.VMEM_SHARED((NPAD,), jnp.float32),
        pltpu.SemaphoreType.DMA,
        pltpu.SemaphoreType.DMA,
    ],
)
def _sc_deg(edge_hbm, ones_hbm, deg_out, i0, i1, ones_v, zv, deg_sh,
            isem0, isem1):
    c = lax.axis_index("c")
    s = lax.axis_index("s")
    w = c * NS + s
    start = DCB * w + jnp.minimum(w, DCH - DCB * NW)
    nch = DCB + jnp.where(w < DCH - DCB * NW, 1, 0)
    end = start + nch

    def _iload(cid, buf, sem):
        pltpu.async_copy(edge_hbm.at[1, pl.ds(cid * 128, 128)], buf, sem)

    def _iwait(cid, buf, sem):
        pltpu.make_async_copy(edge_hbm.at[1, pl.ds(cid * 128, 128)], buf,
                              sem).wait()

    _iload(start, i0, isem0)
    _iload(start + 1, i1, isem1)
    pltpu.sync_copy(ones_hbm, ones_v)
    for j in range(RPT // 16):
        zv[pl.ds(j * 16, 16)] = jnp.zeros((16,), jnp.float32)
    pltpu.sync_copy(zv, deg_sh.at[pl.ds(s * RPT, RPT)])
    plsc.subcore_barrier()

    def body(jj, carry):
        c0 = start + jj * 2
        _iwait(c0, i0, isem0)
        pltpu.sync_copy(ones_v, deg_sh.at[i0], add=True)

        @pl.when(c0 + 2 < end)
        def _():
            _iload(c0 + 2, i0, isem0)

        c1 = c0 + 1
        _iwait(c1, i1, isem1)
        pltpu.sync_copy(ones_v, deg_sh.at[i1], add=True)

        @pl.when(c1 + 2 < end)
        def _():
            _iload(c1 + 2, i1, isem1)

        return carry

    lax.fori_loop(0, DCB // 2, body, 0)

    @pl.when(w < DCH - DCB * NW)
    def _():
        cT = start + DCB
        _iwait(cT, i0, isem0)
        pltpu.sync_copy(ones_v, deg_sh.at[i0], add=True)

    plsc.subcore_barrier()
    pltpu.sync_copy(deg_sh.at[pl.ds(s * RPT, RPT)],
                    deg_out.at[c, pl.ds(s * RPT, RPT)])


@functools.partial(
    pl.kernel,
    mesh=_sc_mesh,
    out_type=jax.ShapeDtypeStruct((NC, NPAD, D), jnp.float32),
    scratch_types=[
        pltpu.VMEM((EPT,), jnp.int32),
        pltpu.VMEM((ROWS_W, EK), jnp.int32),
        pltpu.VMEM((EK, D), jnp.float32),
        pltpu.VMEM((EK, D), jnp.float32),
        pltpu.VMEM_SHARED((NPAD, D), jnp.float32),
        pltpu.SemaphoreType.DMA,
        pltpu.SemaphoreType.DMA,
        pltpu.SemaphoreType.DMA,
        pltpu.SemaphoreType.DMA,
    ],
)
def _sc_scatter(g_hbm, src_hbm, edge_hbm, zeros_hbm, s_out, src_v, dst_v,
                rows0, rows1, s_sh, sem0, sem1, ssem0, ssem1):
    c = lax.axis_index("c")
    s = lax.axis_index("s")
    w = c * NS + s
    HK = EK // 2

    def _gather(cc, rows, sem):
        pltpu.async_copy(g_hbm.at[src_v.at[pl.ds(cc * EK, HK)]],
                         rows.at[pl.ds(0, HK)], sem)
        pltpu.async_copy(g_hbm.at[src_v.at[pl.ds(cc * EK + HK, HK)]],
                         rows.at[pl.ds(HK, HK)], sem)

    def _gwait(cc, rows, sem):
        pltpu.make_async_copy(g_hbm.at[src_v.at[pl.ds(cc * EK, HK)]],
                              rows.at[pl.ds(0, HK)], sem).wait()
        pltpu.make_async_copy(g_hbm.at[src_v.at[pl.ds(cc * EK + HK, HK)]],
                              rows.at[pl.ds(HK, HK)], sem).wait()

    pltpu.sync_copy(src_hbm.at[w], src_v)
    _gather(0, rows0, sem0)
    _gather(1, rows1, sem1)
    pltpu.sync_copy(edge_hbm.at[1, w], dst_v)
    pltpu.sync_copy(zeros_hbm.at[pl.ds(s * RPT, RPT)],
                    s_sh.at[pl.ds(s * RPT, RPT)])
    plsc.subcore_barrier()

    def body(jj, carry):
        c0 = jj * 2
        _gwait(c0, rows0, sem0)
        pltpu.sync_copy(rows0, s_sh.at[dst_v.at[c0]], add=True)

        @pl.when(c0 + 2 < ROWS_W)
        def _():
            _gather(c0 + 2, rows0, sem0)

        c1 = c0 + 1
        _gwait(c1, rows1, sem1)
        pltpu.sync_copy(rows1, s_sh.at[dst_v.at[c1]], add=True)

        @pl.when(c1 + 2 < ROWS_W)
        def _():
            _gather(c1 + 2, rows1, sem1)

        return carry

    lax.fori_loop(0, ROWS_W // 2, body, 0)

    cT = ROWS_W - 1
    _gwait(cT, rows0, sem0)
    pltpu.sync_copy(rows0, s_sh.at[dst_v.at[cT]], add=True)
    plsc.subcore_barrier()
    pltpu.sync_copy(s_sh.at[pl.ds(s * RPT, RPT)],
                    s_out.at[c, pl.ds(s * RPT, RPT)])


def _tc_prep_body(x_ref, w_ref, b_ref, deg_ref, g_ref, dinv_ref):
    h = jnp.dot(x_ref[...], w_ref[...],
                preferred_element_type=jnp.float32) + b_ref[...]
    nrm = jnp.sqrt(jnp.sum(h * h, axis=1, keepdims=True))
    h = h / jnp.maximum(nrm, 1e-12) * 1.8
    deg = deg_ref[0, :N] + deg_ref[1, :N] + 1.0
    dinv = lax.rsqrt(deg)
    dinv_ref[...] = dinv[:, None]
    g_ref[...] = h * dinv[:, None]


_tc_prep = pl.pallas_call(
    _tc_prep_body,
    out_shape=[
        jax.ShapeDtypeStruct((N, D), jnp.float32),
        jax.ShapeDtypeStruct((N, 1), jnp.float32),
    ],
)


def _tc_final_body(s_ref, g_ref, dinv_ref, o_ref):
    ssum = s_ref[0, :N] + s_ref[1, :N]
    o_ref[...] = (ssum + g_ref[...]) * dinv_ref[...]


_tc_final = pl.pallas_call(
    _tc_final_body,
    out_shape=jax.ShapeDtypeStruct((N, D), jnp.float32),
)


def kernel(x, edge_index, W1, b1):
    ei = jnp.asarray(edge_index, jnp.int32)
    e = ei.reshape(2, NW, ROWS_W, EK)
    src_flat = ei[0].reshape(NW, EPT)
    ones = jnp.ones((128,), jnp.float32)
    zeros2 = jnp.zeros((NPAD, D), jnp.float32)
    degp = _sc_deg(ei, ones)
    g, dinv = _tc_prep(x, W1, b1, degp)
    sp = _sc_scatter(g, src_flat, e, zeros2)
    return _tc_final(sp, g, dinv)


# speedup vs baseline: 1.0286x; 1.0286x over previous
import functools

import jax
import jax.numpy as jnp
from jax import lax
from jax.experimental import pallas as pl
from jax.experimental.pallas import tpu as pltpu
from jax.experimental.pallas import tpu_sc as plsc

N = 10000
NPAD = 10240
E = 320000
D = 128
NC = 2
NS = 16
NW = NC * NS
EK = 80
ROWS_W = E // (NW * EK)
EPT = ROWS_W * EK
RPT = NPAD // NS
BN = 400

_sc_mesh = plsc.VectorSubcoreMesh(core_axis_name="c", subcore_axis_name="s")


@functools.partial(
    pl.kernel,
    mesh=_sc_mesh,
    out_type=jax.ShapeDtypeStruct((NC, NPAD), jnp.float32),
    scratch_types=[
        pltpu.VMEM((ROWS_W, EK), jnp.int32),
        pltpu.VMEM((EK,), jnp.float32),
        pltpu.VMEM((RPT,), jnp.float32),
        pltpu.VMEM_SHARED((NPAD,), jnp.float32),
        pltpu.SemaphoreType.DMA,
    ],
)
def _sc_deg(edge_hbm, ones_hbm, deg_out, idx_v, ones_v, zv, deg_sh, dsem):
    c = lax.axis_index("c")
    s = lax.axis_index("s")
    w = c * NS + s
    pltpu.sync_copy(ones_hbm, ones_v)
    for j in range(RPT // 16):
        zv[pl.ds(j * 16, 16)] = jnp.zeros((16,), jnp.float32)
    pltpu.sync_copy(zv, deg_sh.at[pl.ds(s * RPT, RPT)])
    pltpu.sync_copy(edge_hbm.at[1, w], idx_v)
    plsc.subcore_barrier()

    def body(j, carry):
        pltpu.async_copy(ones_v, deg_sh.at[idx_v.at[j]], dsem, add=True)

        @pl.when(j >= 8)
        def _():
            pltpu.make_async_copy(ones_v, deg_sh.at[idx_v.at[j - 8]],
                                  dsem).wait()

        return carry

    lax.fori_loop(0, ROWS_W, body, 0)

    def drain(j, carry):
        pltpu.make_async_copy(ones_v, deg_sh.at[idx_v.at[j]], dsem).wait()
        return carry

    lax.fori_loop(ROWS_W - 8, ROWS_W, drain, 0)
    plsc.subcore_barrier()
    pltpu.sync_copy(deg_sh.at[pl.ds(s * RPT, RPT)],
                    deg_out.at[c, pl.ds(s * RPT, RPT)])


@functools.partial(
    pl.kernel,
    mesh=_sc_mesh,
    out_type=jax.ShapeDtypeStruct((NC, NPAD, D), jnp.float32),
    scratch_types=[
        pltpu.VMEM((EPT,), jnp.int32),
        pltpu.VMEM((ROWS_W, EK), jnp.int32),
        pltpu.VMEM((EK, D), jnp.float32),
        pltpu.VMEM((EK, D), jnp.float32),
        pltpu.VMEM_SHARED((NPAD, D), jnp.float32),
        pltpu.SemaphoreType.DMA,
        pltpu.SemaphoreType.DMA,
        pltpu.SemaphoreType.DMA,
        pltpu.SemaphoreType.DMA,
    ],
)
def _sc_scatter(g_hbm, src_hbm, edge_hbm, zeros_hbm, s_out, src_v, dst_v,
                rows0, rows1, s_sh, sem0, sem1, ssem0, ssem1):
    c = lax.axis_index("c")
    s = lax.axis_index("s")
    w = c * NS + s
    HK = EK // 2

    def _gather(cc, rows, sem):
        pltpu.async_copy(g_hbm.at[src_v.at[pl.ds(cc * EK, HK)]],
                         rows.at[pl.ds(0, HK)], sem)
        pltpu.async_copy(g_hbm.at[src_v.at[pl.ds(cc * EK + HK, HK)]],
                         rows.at[pl.ds(HK, HK)], sem)

    def _gwait(cc, rows, sem):
        pltpu.make_async_copy(g_hbm.at[src_v.at[pl.ds(cc * EK, HK)]],
                              rows.at[pl.ds(0, HK)], sem).wait()
        pltpu.make_async_copy(g_hbm.at[src_v.at[pl.ds(cc * EK + HK, HK)]],
                              rows.at[pl.ds(HK, HK)], sem).wait()

    pltpu.sync_copy(src_hbm.at[w], src_v)
    _gather(0, rows0, sem0)
    _gather(1, rows1, sem1)
    pltpu.sync_copy(edge_hbm.at[1, w], dst_v)
    pltpu.sync_copy(zeros_hbm.at[pl.ds(s * RPT, RPT)],
                    s_sh.at[pl.ds(s * RPT, RPT)])
    plsc.subcore_barrier()

    def body(jj, carry):
        c0 = jj * 2
        _gwait(c0, rows0, sem0)
        pltpu.sync_copy(rows0, s_sh.at[dst_v.at[c0]], add=True)

        @pl.when(c0 + 2 < ROWS_W)
        def _():
            _gather(c0 + 2, rows0, sem0)

        c1 = c0 + 1
        _gwait(c1, rows1, sem1)
        pltpu.sync_copy(rows1, s_sh.at[dst_v.at[c1]], add=True)

        @pl.when(c1 + 2 < ROWS_W)
        def _():
            _gather(c1 + 2, rows1, sem1)

        return carry

    lax.fori_loop(0, ROWS_W // 2, body, 0)

    cT = ROWS_W - 1
    _gwait(cT, rows0, sem0)
    pltpu.sync_copy(rows0, s_sh.at[dst_v.at[cT]], add=True)
    plsc.subcore_barrier()
    pltpu.sync_copy(s_sh.at[pl.ds(s * RPT, RPT)],
                    s_out.at[c, pl.ds(s * RPT, RPT)])


def _tc_prep_body(x_ref, w_ref, b_ref, deg_ref, g_ref, dinv_ref):
    h = jnp.dot(x_ref[...], w_ref[...],
                preferred_element_type=jnp.float32) + b_ref[...]
    nrm = jnp.sqrt(jnp.sum(h * h, axis=1, keepdims=True))
    h = h / jnp.maximum(nrm, 1e-12) * 1.8
    deg = deg_ref[0, :N] + deg_ref[1, :N] + 1.0
    dinv = lax.rsqrt(deg)
    dinv_ref[...] = dinv[:, None]
    g_ref[...] = h * dinv[:, None]


_tc_prep = pl.pallas_call(
    _tc_prep_body,
    out_shape=[
        jax.ShapeDtypeStruct((N, D), jnp.float32),
        jax.ShapeDtypeStruct((N, 1), jnp.float32),
    ],
)


def _tc_final_body(s_ref, g_ref, dinv_ref, o_ref):
    ssum = s_ref[0, :N] + s_ref[1, :N]
    o_ref[...] = (ssum + g_ref[...]) * dinv_ref[...]


_tc_final = pl.pallas_call(
    _tc_final_body,
    out_shape=jax.ShapeDtypeStruct((N, D), jnp.float32),
)


def kernel(x, edge_index, W1, b1):
    ei = jnp.asarray(edge_index, jnp.int32)
    e = ei.reshape(2, NW, ROWS_W, EK)
    src_flat = ei[0].reshape(NW, EPT)
    ones = jnp.ones((EK,), jnp.float32)
    zeros2 = jnp.zeros((NPAD, D), jnp.float32)
    degp = _sc_deg(e, ones)
    g, dinv = _tc_prep(x, W1, b1, degp)
    sp = _sc_scatter(g, src_flat, e, zeros2)
    return _tc_final(sp, g, dinv)
